# baseline (speedup 1.0000x reference)
import jax
import jax.numpy as jnp
from jax.experimental import pallas as pl
from jax.experimental.pallas import tpu as pltpu
from jax.experimental.pallas import tpu_sc as plsc

_WIN = 128


def kernel(table, words_pretrained):
    D = table.shape[1]
    B, L = words_pretrained.shape
    N = B * L
    assert N % _WIN == 0
    idx = words_pretrained.reshape(1, N).astype(jnp.int32)

    mesh = plsc.VectorSubcoreMesh(
        core_axis_name="core", subcore_axis_name="subcore"
    )

    @jax.jit
    def run(table, idx):
        @pl.kernel(
            out_type=jax.ShapeDtypeStruct((N, D), table.dtype),
            mesh=mesh,
            compiler_params=pltpu.CompilerParams(use_tc_tiling_on_sc=False),
        )
        def k(x_hbm, i_hbm, o_hbm):
            def body(i_vmem, o_vmem):
                pltpu.sync_copy(x_hbm.at[i_vmem.at[0]], o_vmem)

            pltpu.emit_pipeline(
                body,
                grid=(N // _WIN,),
                in_specs=[
                    pl.BlockSpec((1, _WIN), index_map=lambda i: (0, i))
                ],
                out_specs=[
                    pl.BlockSpec((_WIN, D), index_map=lambda i: (i, 0))
                ],
                core_axis_name=("core", "subcore"),
                dimension_semantics=(pltpu.PARALLEL,),
            )(i_hbm, o_hbm)

        return k(table, idx)

    return run(table, idx).reshape(B, L, D)


# speedup vs baseline: 1.0733x; 1.0733x over previous
import jax
import jax.numpy as jnp
from jax.experimental import pallas as pl
from jax.experimental.pallas import tpu as pltpu
from jax.experimental.pallas import tpu_sc as plsc

_WIN = 128
_NW = 32
_SLOTS = 8


def kernel(table, words_pretrained):
    D = table.shape[1]
    B, L = words_pretrained.shape
    N = B * L
    assert N % (_NW * _WIN) == 0
    nch = N // (_NW * _WIN)
    assert nch % _SLOTS == 0
    idx = words_pretrained.reshape(_NW, nch, _WIN).astype(jnp.int32)

    mesh = plsc.VectorSubcoreMesh(
        core_axis_name="core", subcore_axis_name="subcore"
    )

    @jax.jit
    def run(table, idx):
        @pl.kernel(
            out_type=jax.ShapeDtypeStruct((N, D), table.dtype),
            mesh=mesh,
            compiler_params=pltpu.CompilerParams(use_tc_tiling_on_sc=False),
            scratch_types=[
                pltpu.VMEM((nch, _WIN), jnp.int32),
                pltpu.VMEM((_SLOTS * _WIN, D), table.dtype),
                pltpu.SemaphoreType.DMA,
                pltpu.SemaphoreType.DMA((_SLOTS,)),
                pltpu.SemaphoreType.DMA((_SLOTS,)),
            ],
        )
        def k(x_hbm, i_hbm, o_hbm, idx_v, rows_v, isem, gsem, ssem):
            wid = (
                jax.lax.axis_index("core") * 16
                + jax.lax.axis_index("subcore")
            )
            base = wid * nch * _WIN
            pltpu.async_copy(i_hbm.at[wid], idx_v, isem).wait()

            def gather(c, b):
                pltpu.async_copy(
                    x_hbm.at[idx_v.at[c]],
                    rows_v.at[pl.ds(b * _WIN, _WIN)],
                    gsem.at[b],
                )

            def store(c, b):
                pltpu.async_copy(
                    rows_v.at[pl.ds(b * _WIN, _WIN)],
                    o_hbm.at[pl.ds(base + c * _WIN, _WIN)],
                    ssem.at[b],
                )

            for b in range(_SLOTS):
                gather(b, b)

            @pl.loop(0, nch, step=_SLOTS)
            def _(c):
                for b in range(_SLOTS):
                    pltpu.make_async_copy(
                        x_hbm.at[idx_v.at[c + b]],
                        rows_v.at[pl.ds(b * _WIN, _WIN)],
                        gsem.at[b],
                    ).wait()
                    store(c + b, b)
                for b in range(_SLOTS):
                    @pl.when(c + _SLOTS + b < nch)
                    def _():
                        pltpu.make_async_copy(
                            rows_v.at[pl.ds(b * _WIN, _WIN)],
                            o_hbm.at[pl.ds(base, _WIN)],
                            ssem.at[b],
                        ).wait()
                        gather(c + _SLOTS + b, b)

            for b in range(_SLOTS):
                pltpu.make_async_copy(
                    rows_v.at[pl.ds(b * _WIN, _WIN)],
                    o_hbm.at[pl.ds(base, _WIN)],
                    ssem.at[b],
                ).wait()

        return k(table, idx)

    return run(table, idx).reshape(B, L, D)
